# baseline (speedup 1.0000x reference)
import jax
import jax.numpy as jnp
from jax.experimental import pallas as pl


def _pos_kernel(row_ref, col_ref, out_ref):
    h = row_ref.shape[0]
    w = col_ref.shape[0]
    f_half = row_ref.shape[1]
    col_t = jnp.transpose(col_ref[...], (1, 0))
    out_ref[0, 0:f_half, :, :] = jnp.broadcast_to(
        col_t[:, None, :], (f_half, h, w)
    )
    row_t = jnp.transpose(row_ref[...], (1, 0))
    out_ref[0, f_half:2 * f_half, :, :] = jnp.broadcast_to(
        row_t[:, :, None], (f_half, h, w)
    )


def kernel(mask, row_embed, col_embed):
    b, h, w = mask.shape
    f_half = row_embed.shape[1]
    f = 2 * f_half
    out = pl.pallas_call(
        _pos_kernel,
        grid=(b,),
        in_specs=[
            pl.BlockSpec((h, f_half), lambda i: (0, 0)),
            pl.BlockSpec((w, f_half), lambda i: (0, 0)),
        ],
        out_specs=pl.BlockSpec((1, f, h, w), lambda i: (i, 0, 0, 0)),
        out_shape=jax.ShapeDtypeStruct((b, f, h, w), jnp.float32),
    )(row_embed, col_embed)
    return out


# speedup vs baseline: 1.0011x; 1.0011x over previous
import jax
import jax.numpy as jnp
from jax.experimental import pallas as pl
from jax.experimental.pallas import tpu as pltpu


def _pos_kernel(row_ref, col_ref, out_ref, scratch, sem):
    h = row_ref.shape[0]
    w = col_ref.shape[0]
    f_half = row_ref.shape[1]
    col_t = jnp.transpose(col_ref[...], (1, 0))
    scratch[0:f_half, :, :] = jnp.broadcast_to(col_t[:, None, :], (f_half, h, w))
    row_t = jnp.transpose(row_ref[...], (1, 0))
    scratch[f_half:2 * f_half, :, :] = jnp.broadcast_to(row_t[:, :, None], (f_half, h, w))
    b = out_ref.shape[0]
    copies = [pltpu.make_async_copy(scratch, out_ref.at[i], sem) for i in range(b)]
    for c in copies:
        c.start()
    for c in copies:
        c.wait()


def kernel(mask, row_embed, col_embed):
    b, h, w = mask.shape
    f_half = row_embed.shape[1]
    f = 2 * f_half
    out = pl.pallas_call(
        _pos_kernel,
        out_specs=pl.BlockSpec(memory_space=pl.ANY),
        out_shape=jax.ShapeDtypeStruct((b, f, h, w), jnp.float32),
        scratch_shapes=[
            pltpu.VMEM((f, h, w), jnp.float32),
            pltpu.SemaphoreType.DMA,
        ],
    )(row_embed, col_embed)
    return out


# speedup vs baseline: 1.6295x; 1.6276x over previous
import jax
import jax.numpy as jnp
from jax.experimental import pallas as pl
from jax.experimental.pallas import tpu as pltpu


def _pos_kernel(row_ref, col_ref, out_ref, scratch, sem):
    h = row_ref.shape[0]
    w = col_ref.shape[0]
    f_half = row_ref.shape[1]
    col_t = jnp.transpose(col_ref[...], (1, 0))
    row_t = jnp.transpose(row_ref[...], (1, 0))
    for i in range(h):
        scratch[0:f_half, i * w:(i + 1) * w] = col_t
        scratch[f_half:2 * f_half, i * w:(i + 1) * w] = jnp.broadcast_to(
            row_t[:, i:i + 1], (f_half, w)
        )
    b = out_ref.shape[0]
    copies = [pltpu.make_async_copy(scratch, out_ref.at[i], sem) for i in range(b)]
    for c in copies:
        c.start()
    for c in copies:
        c.wait()


def kernel(mask, row_embed, col_embed):
    b, h, w = mask.shape
    f_half = row_embed.shape[1]
    f = 2 * f_half
    out = pl.pallas_call(
        _pos_kernel,
        out_specs=pl.BlockSpec(memory_space=pl.ANY),
        out_shape=jax.ShapeDtypeStruct((b, f, h * w), jnp.float32),
        scratch_shapes=[
            pltpu.VMEM((f, h * w), jnp.float32),
            pltpu.SemaphoreType.DMA,
        ],
    )(row_embed, col_embed)
    return out.reshape(b, f, h, w)


# speedup vs baseline: 1.6315x; 1.0013x over previous
import jax
import jax.numpy as jnp
from jax.experimental import pallas as pl
from jax.experimental.pallas import tpu as pltpu


def _pos_kernel(row_ref, col_ref, out_ref, scratch, sem):
    h = row_ref.shape[0]
    w = col_ref.shape[0]
    f_half = row_ref.shape[1]
    col_t = jnp.transpose(col_ref[...], (1, 0))
    row_t = jnp.transpose(row_ref[...], (1, 0))
    for i in range(h):
        scratch[0:f_half, i * w:(i + 1) * w] = col_t
        scratch[f_half:2 * f_half, i * w:(i + 1) * w] = jnp.broadcast_to(
            row_t[:, i:i + 1], (f_half, w)
        )
    b = out_ref.shape[0]
    copies = [pltpu.make_async_copy(scratch, out_ref.at[i], sem.at[i]) for i in range(b)]
    for c in copies:
        c.start()
    for c in copies:
        c.wait()


def kernel(mask, row_embed, col_embed):
    b, h, w = mask.shape
    f_half = row_embed.shape[1]
    f = 2 * f_half
    out = pl.pallas_call(
        _pos_kernel,
        out_specs=pl.BlockSpec(memory_space=pl.ANY),
        out_shape=jax.ShapeDtypeStruct((b, f, h * w), jnp.float32),
        scratch_shapes=[
            pltpu.VMEM((f, h * w), jnp.float32),
            pltpu.SemaphoreType.DMA((32,)),
        ],
    )(row_embed, col_embed)
    return out.reshape(b, f, h, w)
